# Initial kernel scaffold; baseline (speedup 1.0000x reference)
import jax
import jax.numpy as jnp
from jax.experimental import pallas as pl


def kernel(xyz):
    raise NotImplementedError("write your pallas kernel here")


# speedup vs baseline: 4.6418x; 4.6418x over previous
import jax
import jax.numpy as jnp
from jax.experimental import pallas as pl

_G = 512
_K = 32
_NL = 128


def _fps_body(xp_ref, cen_ref):
    b, _, nr, nl = xp_ref.shape
    x = xp_ref[:, 0]
    y = xp_ref[:, 1]
    z = xp_ref[:, 2]
    riota = jax.lax.broadcasted_iota(jnp.int32, (b, nr, nl), 1)
    liota = jax.lax.broadcasted_iota(jnp.int32, (b, nr, nl), 2)
    niota = riota * nl + liota
    g_iota = jax.lax.broadcasted_iota(jnp.int32, (b, _G), 1)

    lx = x[:, 0:1, 0:1]
    ly = y[:, 0:1, 0:1]
    lz = z[:, 0:1, 0:1]
    cx = jnp.where(g_iota == 0, lx[:, :, 0], 0.0)
    cy = jnp.where(g_iota == 0, ly[:, :, 0], 0.0)
    cz = jnp.where(g_iota == 0, lz[:, :, 0], 0.0)
    dists = jnp.full((b, nr, nl), 1e10, jnp.float32)

    def body(i, st):
        dists, lx, ly, lz, cx, cy, cz = st
        dx = x - lx
        dy = y - ly
        dz = z - lz
        d = dx * dx + dy * dy + dz * dz
        dists = jnp.minimum(dists, d)
        m = jnp.max(jnp.max(dists, axis=2, keepdims=True), axis=1,
                    keepdims=True)
        wi = jnp.where(dists == m, niota, jnp.int32(2 ** 30))
        nxt = jnp.min(jnp.min(wi, axis=2, keepdims=True), axis=1,
                      keepdims=True)
        sel = niota == nxt
        nlx = jnp.sum(jnp.sum(jnp.where(sel, x, 0.0), axis=2, keepdims=True),
                      axis=1, keepdims=True)
        nly = jnp.sum(jnp.sum(jnp.where(sel, y, 0.0), axis=2, keepdims=True),
                      axis=1, keepdims=True)
        nlz = jnp.sum(jnp.sum(jnp.where(sel, z, 0.0), axis=2, keepdims=True),
                      axis=1, keepdims=True)
        oh = g_iota == i
        cx = cx + jnp.where(oh, nlx[:, :, 0], 0.0)
        cy = cy + jnp.where(oh, nly[:, :, 0], 0.0)
        cz = cz + jnp.where(oh, nlz[:, :, 0], 0.0)
        return dists, nlx, nly, nlz, cx, cy, cz

    st = jax.lax.fori_loop(1, _G, body, (dists, lx, ly, lz, cx, cy, cz))
    cen_ref[:, 0] = st[4]
    cen_ref[:, 1] = st[5]
    cen_ref[:, 2] = st[6]


def _knn_body(xr_ref, ct_ref, nb_ref):
    n = xr_ref.shape[2]
    x = xr_ref[0, 0:1, :]
    y = xr_ref[0, 1:2, :]
    z = xr_ref[0, 2:3, :]
    cx = ct_ref[0, :, 0:1]
    cy = ct_ref[0, :, 1:2]
    cz = ct_ref[0, :, 2:3]
    pn = x * x + y * y + z * z
    cn = cx * cx + cy * cy + cz * cz
    xb = x.astype(jnp.bfloat16).astype(jnp.float32)
    yb = y.astype(jnp.bfloat16).astype(jnp.float32)
    zb = z.astype(jnp.bfloat16).astype(jnp.float32)
    cxb = cx.astype(jnp.bfloat16).astype(jnp.float32)
    cyb = cy.astype(jnp.bfloat16).astype(jnp.float32)
    czb = cz.astype(jnp.bfloat16).astype(jnp.float32)
    p0 = cxb * xb
    p1 = cyb * yb
    p2 = czb * zb
    s1 = p0 + p1
    ap = s1 - p1
    e1 = (p0 - ap) + (p1 - (s1 - ap))
    s2 = s1 + p2
    bp = s2 - p2
    e2 = (s1 - bp) + (p2 - (s2 - bp))
    cross = s2 + (e1 + e2)
    d2 = (cn + pn) - 2.0 * cross
    liota = jax.lax.broadcasted_iota(jnp.int32, (_G, n), 1)
    kiota = jax.lax.broadcasted_iota(jnp.int32, (_G, _K), 1)
    zeros = jnp.zeros((_G, _K), jnp.float32)

    def body(i, st):
        dmat, nbx, nby, nbz = st
        m = jnp.min(dmat, axis=1, keepdims=True)
        wi = jnp.where(dmat == m, liota, jnp.int32(2 ** 30))
        idx = jnp.min(wi, axis=1, keepdims=True)
        first = liota == idx
        gx = jnp.sum(jnp.where(first, x, 0.0), axis=1, keepdims=True)
        gy = jnp.sum(jnp.where(first, y, 0.0), axis=1, keepdims=True)
        gz = jnp.sum(jnp.where(first, z, 0.0), axis=1, keepdims=True)
        dmat = jnp.where(first, jnp.float32(jnp.inf), dmat)
        ohk = kiota == i
        nbx = nbx + jnp.where(ohk, gx - cx, 0.0)
        nby = nby + jnp.where(ohk, gy - cy, 0.0)
        nbz = nbz + jnp.where(ohk, gz - cz, 0.0)
        return dmat, nbx, nby, nbz

    _, nbx, nby, nbz = jax.lax.fori_loop(0, _K, body,
                                         (d2, zeros, zeros, zeros))
    nb_ref[0, 0] = nbx
    nb_ref[0, 1] = nby
    nb_ref[0, 2] = nbz


def kernel(xyz):
    b, n, c = xyz.shape
    nr = n // _NL
    xt = jnp.transpose(xyz, (0, 2, 1))
    xp = xt.reshape(b, 3, nr, _NL)
    cen = pl.pallas_call(
        _fps_body,
        out_shape=jax.ShapeDtypeStruct((b, 3, _G), jnp.float32),
    )(xp)
    center = jnp.transpose(cen, (0, 2, 1))
    nb = pl.pallas_call(
        _knn_body,
        grid=(b,),
        in_specs=[
            pl.BlockSpec((1, 3, n), lambda i: (i, 0, 0)),
            pl.BlockSpec((1, _G, 3), lambda i: (i, 0, 0)),
        ],
        out_specs=pl.BlockSpec((1, 3, _G, _K), lambda i: (i, 0, 0, 0)),
        out_shape=jax.ShapeDtypeStruct((b, 3, _G, _K), jnp.float32),
    )(xt, center)
    neighborhood = jnp.transpose(nb, (0, 2, 3, 1))
    return neighborhood, center


# speedup vs baseline: 6.7044x; 1.4443x over previous
import functools

import jax
import jax.numpy as jnp
from jax.experimental import pallas as pl
from jax.experimental.pallas import tpu as pltpu
from jax.experimental.pallas import tpu_sc as plsc

_G = 512
_K = 32
_NL = 128
_NC = 2
_NS = 16
_PW = 8192
_CHUNK = 2048


def _fps_body(xp_ref, cen_ref):
    b, _, nr, nl = xp_ref.shape
    x = xp_ref[:, 0]
    y = xp_ref[:, 1]
    z = xp_ref[:, 2]
    riota = jax.lax.broadcasted_iota(jnp.int32, (b, nr, nl), 1)
    liota = jax.lax.broadcasted_iota(jnp.int32, (b, nr, nl), 2)
    niota = riota * nl + liota
    g_iota = jax.lax.broadcasted_iota(jnp.int32, (b, _G), 1)

    lx = x[:, 0:1, 0:1]
    ly = y[:, 0:1, 0:1]
    lz = z[:, 0:1, 0:1]
    cx = jnp.where(g_iota == 0, lx[:, :, 0], 0.0)
    cy = jnp.where(g_iota == 0, ly[:, :, 0], 0.0)
    cz = jnp.where(g_iota == 0, lz[:, :, 0], 0.0)
    dists = jnp.full((b, nr, nl), 1e10, jnp.float32)

    def body(i, st):
        dists, lx, ly, lz, cx, cy, cz = st
        dx = x - lx
        dy = y - ly
        dz = z - lz
        d = dx * dx + dy * dy + dz * dz
        dists = jnp.minimum(dists, d)
        m = jnp.max(jnp.max(dists, axis=2, keepdims=True), axis=1,
                    keepdims=True)
        wi = jnp.where(dists == m, niota, jnp.int32(2 ** 30))
        nxt = jnp.min(jnp.min(wi, axis=2, keepdims=True), axis=1,
                      keepdims=True)
        sel = niota == nxt
        nlx = jnp.sum(jnp.sum(jnp.where(sel, x, 0.0), axis=2, keepdims=True),
                      axis=1, keepdims=True)
        nly = jnp.sum(jnp.sum(jnp.where(sel, y, 0.0), axis=2, keepdims=True),
                      axis=1, keepdims=True)
        nlz = jnp.sum(jnp.sum(jnp.where(sel, z, 0.0), axis=2, keepdims=True),
                      axis=1, keepdims=True)
        oh = g_iota == i
        cx = cx + jnp.where(oh, nlx[:, :, 0], 0.0)
        cy = cy + jnp.where(oh, nly[:, :, 0], 0.0)
        cz = cz + jnp.where(oh, nlz[:, :, 0], 0.0)
        return dists, nlx, nly, nlz, cx, cy, cz

    st = jax.lax.fori_loop(1, _G, body, (dists, lx, ly, lz, cx, cy, cz))
    cen_ref[:, 0] = st[4]
    cen_ref[:, 1] = st[5]
    cen_ref[:, 2] = st[6]


def _knn_body(xr_ref, ct_ref, nb_ref):
    n = xr_ref.shape[2]
    x = xr_ref[0, 0:1, :]
    y = xr_ref[0, 1:2, :]
    z = xr_ref[0, 2:3, :]
    cx = ct_ref[0, :, 0:1]
    cy = ct_ref[0, :, 1:2]
    cz = ct_ref[0, :, 2:3]
    pn = x * x + y * y + z * z
    cn = cx * cx + cy * cy + cz * cz
    xb = x.astype(jnp.bfloat16).astype(jnp.float32)
    yb = y.astype(jnp.bfloat16).astype(jnp.float32)
    zb = z.astype(jnp.bfloat16).astype(jnp.float32)
    cxb = cx.astype(jnp.bfloat16).astype(jnp.float32)
    cyb = cy.astype(jnp.bfloat16).astype(jnp.float32)
    czb = cz.astype(jnp.bfloat16).astype(jnp.float32)
    p0 = cxb * xb
    p1 = cyb * yb
    p2 = czb * zb
    s1 = p0 + p1
    ap = s1 - p1
    e1 = (p0 - ap) + (p1 - (s1 - ap))
    s2 = s1 + p2
    bp = s2 - p2
    e2 = (s1 - bp) + (p2 - (s2 - bp))
    cross = s2 + (e1 + e2)
    d2 = (cn + pn) - 2.0 * cross
    liota = jax.lax.broadcasted_iota(jnp.int32, (_G, n), 1)
    kiota = jax.lax.broadcasted_iota(jnp.int32, (_G, _K), 1)

    def body(i, st):
        dmat, acc = st
        m = jnp.min(dmat, axis=1, keepdims=True)
        wi = jnp.where(dmat == m, liota, jnp.int32(2 ** 30))
        idx = jnp.min(wi, axis=1, keepdims=True)
        first = liota == idx
        dmat = jnp.where(first, jnp.float32(jnp.inf), dmat)
        acc = acc + jnp.where(kiota == i, idx, 0)
        return dmat, acc

    _, acc = jax.lax.fori_loop(0, _K, body,
                               (d2, jnp.zeros((_G, _K), jnp.int32)))
    nb_ref[0] = acc + pl.program_id(0) * n


def _sc_gather_body(tbl, ctbn, idxs, out, idx_v, rows_v, crows_v, sem):
    wid = jax.lax.axis_index("s") * _NC + jax.lax.axis_index("c")
    for t in range(_PW // _CHUNK):
        base = pl.multiple_of(wid * _PW + t * _CHUNK, _CHUNK)
        pltpu.sync_copy(idxs.at[pl.ds(base, _CHUNK)], idx_v)
        pltpu.async_copy(tbl.at[idx_v], rows_v, sem).wait()
        cbase = pl.multiple_of(base // _K, _CHUNK // _K)
        pltpu.sync_copy(ctbn.at[pl.ds(cbase, _CHUNK // _K)], crows_v)

        def sub_step(r, _):
            rows_v[r] = rows_v[r] + crows_v[jax.lax.shift_right_logical(
                r, 5)]
            return 0

        jax.lax.fori_loop(0, _CHUNK, sub_step, 0)
        pltpu.sync_copy(rows_v, out.at[pl.ds(base, _CHUNK)])


def kernel(xyz):
    b, n, c = xyz.shape
    nr = n // _NL
    xt = jnp.transpose(xyz, (0, 2, 1))
    xp = xt.reshape(b, 3, nr, _NL)
    cen = pl.pallas_call(
        _fps_body,
        out_shape=jax.ShapeDtypeStruct((b, 3, _G), jnp.float32),
    )(xp)
    center = jnp.transpose(cen, (0, 2, 1))
    ix = pl.pallas_call(
        _knn_body,
        grid=(b,),
        in_specs=[
            pl.BlockSpec((1, 3, n), lambda i: (i, 0, 0)),
            pl.BlockSpec((1, _G, 3), lambda i: (i, 0, 0)),
        ],
        out_specs=pl.BlockSpec((1, _G, _K), lambda i: (i, 0, 0)),
        out_shape=jax.ShapeDtypeStruct((b, _G, _K), jnp.int32),
    )(xt, center)

    pad = jnp.zeros((b * n, 13), jnp.float32)
    tbl = jnp.concatenate([xyz.reshape(b * n, c), pad], axis=1)
    cpad = jnp.zeros((b * _G, 13), jnp.float32)
    ctbn = jnp.concatenate([-center.reshape(b * _G, c), cpad], axis=1)
    idxf = ix.reshape(-1)

    mesh = plsc.VectorSubcoreMesh(core_axis_name="c", subcore_axis_name="s")
    nb16 = pl.kernel(
        _sc_gather_body,
        out_type=jax.ShapeDtypeStruct((b * _G * _K, 16), jnp.float32),
        mesh=mesh,
        compiler_params=pltpu.CompilerParams(use_tc_tiling_on_sc=False),
        scratch_types=[
            pltpu.VMEM((_CHUNK,), jnp.int32),
            pltpu.VMEM((_CHUNK, 16), jnp.float32),
            pltpu.VMEM((_CHUNK // _K, 16), jnp.float32),
            pltpu.SemaphoreType.DMA,
        ],
    )(tbl, ctbn, idxf)
    neighborhood = nb16[:, :3].reshape(b, _G, _K, 3)
    return neighborhood, center
